# Initial kernel scaffold; baseline (speedup 1.0000x reference)
import jax
import jax.numpy as jnp
from jax.experimental import pallas as pl


def kernel(x, edge_index, batch, W1, b1, W2, b2, W3, b3, Wl, bl):
    raise NotImplementedError("write your pallas kernel here")


# speedup vs baseline: 2.4900x; 2.4900x over previous
import functools

import jax
import jax.numpy as jnp
from jax.experimental import pallas as pl
from jax.experimental.pallas import tpu as pltpu

N_NODES = 50000
N_GRAPHS = 512
ROW_BLK = 2000


def _mm_relu_body(h_ref, w_ref, b_ref, o_ref):
    o_ref[...] = jax.nn.relu(
        jnp.dot(h_ref[...], w_ref[...], preferred_element_type=jnp.float32)
        + b_ref[...]
    )


def _mm_relu(h, W, b):
    n, fin = h.shape
    fout = W.shape[1]
    grid = n // ROW_BLK
    return pl.pallas_call(
        _mm_relu_body,
        grid=(grid,),
        in_specs=[
            pl.BlockSpec((ROW_BLK, fin), lambda i: (i, 0)),
            pl.BlockSpec((fin, fout), lambda i: (0, 0)),
            pl.BlockSpec((1, fout), lambda i: (0, 0)),
        ],
        out_specs=pl.BlockSpec((ROW_BLK, fout), lambda i: (i, 0)),
        out_shape=jax.ShapeDtypeStruct((n, fout), jnp.float32),
    )(h, W, b.reshape(1, fout))


def _propagate(h, src, dst, dinv):
    hs = h * dinv[:, None]
    agg = jnp.zeros_like(h).at[dst].add(hs[src], mode="drop")
    return dinv[:, None] * agg + (dinv * dinv)[:, None] * h


def kernel(x, edge_index, batch, W1, b1, W2, b2, W3, b3, Wl, bl):
    src = edge_index[0]
    dst = edge_index[1]
    deg = jnp.ones((N_NODES,), jnp.float32).at[dst].add(1.0, mode="drop")
    dinv = jax.lax.rsqrt(deg)

    xp = jnp.pad(x, ((0, 0), (0, 3)))
    W1p = jnp.pad(W1, ((0, 3), (0, 0)))
    g1 = _propagate(xp, src, dst, dinv)
    h1 = _mm_relu(g1, W1p, b1)

    g2 = _propagate(h1, src, dst, dinv)
    h2 = _mm_relu(g2, W2, b2)

    g3 = _propagate(h2, src, dst, dinv)
    h3 = _mm_relu(g3, W3, b3)

    sums = jax.ops.segment_sum(h3, batch, num_segments=N_GRAPHS)
    cnt = jax.ops.segment_sum(jnp.ones((N_NODES,), jnp.float32), batch,
                              num_segments=N_GRAPHS)
    pooled = sums / jnp.maximum(cnt, 1.0)[:, None]
    logits = pooled @ Wl + bl
    return jax.nn.log_softmax(logits, axis=1)


# speedup vs baseline: 42.8074x; 17.1917x over previous
import functools

import jax
import jax.numpy as jnp
from jax import lax
from jax.experimental import pallas as pl
from jax.experimental.pallas import tpu as pltpu
from jax.experimental.pallas import tpu_sc as plsc

N_NODES = 50000
N_EDGES = 3200000
N_GRAPHS = 512

SUB = 128
NSUB = 8
CHUNK = SUB * NSUB
NW = 32
G_ITERS = 98
E_PAD = NW * CHUNK * G_ITERS
N_PAD = 50048
ROWS_TBL = N_PAD
ROWS_PER_TILE = 3128

ROW_BLK = 2000
TC_GRID = N_NODES // ROW_BLK


def _sc_body(do_gather, hs, src2, dst2, out, idx_s, idx_d, rows, shared,
             sem_g):
    c = lax.axis_index("c")
    s = lax.axis_index("s")
    w = s * 2 + c
    r0 = s * ROWS_PER_TILE
    pltpu.sync_copy(hs.at[pl.ds(r0, ROWS_PER_TILE)],
                    shared.at[pl.ds(r0, ROWS_PER_TILE)])
    if not do_gather:
        pltpu.sync_copy(hs.at[pl.ds(0, CHUNK)], rows)
    plsc.subcore_barrier()

    base = w * G_ITERS * NSUB

    def body(g, carry):
        row = base + g * NSUB
        pltpu.sync_copy(dst2.at[pl.ds(row, NSUB)], idx_d)
        if do_gather:
            pltpu.sync_copy(src2.at[pl.ds(row, NSUB)], idx_s)
            descs = [
                pltpu.async_copy(hs.at[idx_s.at[j]],
                                 rows.at[pl.ds(j * SUB, SUB)], sem_g)
                for j in range(NSUB)
            ]
            for d in descs:
                d.wait()
        for j in range(NSUB):
            pltpu.sync_copy(rows.at[pl.ds(j * SUB, SUB)],
                            shared.at[idx_d.at[j]], add=True)
        return carry

    lax.fori_loop(0, G_ITERS, body, 0)
    plsc.subcore_barrier()
    pltpu.sync_copy(shared.at[pl.ds(r0, ROWS_PER_TILE)],
                    out.at[c].at[pl.ds(r0, ROWS_PER_TILE)])


@functools.partial(jax.jit, static_argnames=("fsc", "do_gather"))
def _sc_prop(hs, src2, dst2, fsc, do_gather=True):
    mesh = plsc.VectorSubcoreMesh(core_axis_name="c", subcore_axis_name="s")
    return pl.kernel(
        functools.partial(_sc_body, do_gather),
        out_type=jax.ShapeDtypeStruct((2, ROWS_TBL, fsc), jnp.float32),
        mesh=mesh,
        scratch_types=[
            pltpu.VMEM((NSUB, SUB), jnp.int32),
            pltpu.VMEM((NSUB, SUB), jnp.int32),
            pltpu.VMEM((CHUNK, fsc), jnp.float32),
            pltpu.VMEM_SHARED((ROWS_TBL, fsc), jnp.float32),
            pltpu.SemaphoreType.DMA,
        ],
        compiler_params=pltpu.CompilerParams(use_tc_tiling_on_sc=False),
    )(hs, src2, dst2)


def _prep_body(pdeg_ref, xp_ref, dinv_ref, hs0_ref):
    deg = pdeg_ref[0, :, 0:1] + pdeg_ref[1, :, 0:1] - 1.0
    dinv = lax.rsqrt(deg)
    dinv_ref[...] = dinv
    hs0_ref[...] = xp_ref[...] * dinv


def _tc_prep(pdeg, xp):
    return pl.pallas_call(
        _prep_body,
        grid=(TC_GRID,),
        in_specs=[
            pl.BlockSpec((2, ROW_BLK, 16), lambda i: (0, i, 0)),
            pl.BlockSpec((ROW_BLK, 16), lambda i: (i, 0)),
        ],
        out_specs=[
            pl.BlockSpec((ROW_BLK, 1), lambda i: (i, 0)),
            pl.BlockSpec((ROW_BLK, 16), lambda i: (i, 0)),
        ],
        out_shape=[
            jax.ShapeDtypeStruct((N_NODES, 1), jnp.float32),
            jax.ShapeDtypeStruct((N_PAD, 16), jnp.float32),
        ],
    )(pdeg, xp)


def _layer_body(nout, p_ref, hs_ref, dinv_ref, w_ref, b_ref, *o_refs):
    dinv = dinv_ref[...]
    g = dinv * (p_ref[0] + p_ref[1] - hs_ref[...])
    h = jax.nn.relu(
        jnp.dot(g, w_ref[...], preferred_element_type=jnp.float32)
        + b_ref[...])
    hs_next = h * dinv
    if nout == 1:
        o_refs[0][...] = hs_next
    else:
        for q in range(nout):
            o_refs[q][...] = hs_next[:, 16 * q:16 * (q + 1)]


def _tc_layer(p, hs, dinv, W, b, split):
    fsc = hs.shape[1]
    fout = W.shape[1]
    if split:
        nout = fout // 16
        out_specs = [pl.BlockSpec((ROW_BLK, 16), lambda i: (i, 0))] * nout
        out_shape = [jax.ShapeDtypeStruct((N_PAD, 16), jnp.float32)] * nout
    else:
        nout = 1
        out_specs = [pl.BlockSpec((ROW_BLK, fout), lambda i: (i, 0))]
        out_shape = [jax.ShapeDtypeStruct((N_PAD, fout), jnp.float32)]
    return pl.pallas_call(
        functools.partial(_layer_body, nout),
        grid=(TC_GRID,),
        in_specs=[
            pl.BlockSpec((2, ROW_BLK, fsc), lambda i: (0, i, 0)),
            pl.BlockSpec((ROW_BLK, fsc), lambda i: (i, 0)),
            pl.BlockSpec((ROW_BLK, 1), lambda i: (i, 0)),
            pl.BlockSpec((fsc, fout), lambda i: (0, 0)),
            pl.BlockSpec((1, fout), lambda i: (0, 0)),
        ],
        out_specs=out_specs,
        out_shape=out_shape,
    )(p, hs, dinv, W, b.reshape(1, fout))


def _layer3_body(p0_ref, p1_ref, p2_ref, p3_ref, hs0_ref, hs1_ref, hs2_ref,
                 hs3_ref, dinv_ref, w_ref, b_ref,
                 batch_ref, sums_ref, cnt_ref):
    i = pl.program_id(0)
    dinv = dinv_ref[...]
    w = w_ref[...]
    p_refs = (p0_ref, p1_ref, p2_ref, p3_ref)
    hs_refs = (hs0_ref, hs1_ref, hs2_ref, hs3_ref)
    acc = b_ref[...]
    for q in range(4):
        gq = dinv * (p_refs[q][0] + p_refs[q][1] - hs_refs[q][...])
        acc = acc + jnp.dot(gq, w[16 * q:16 * (q + 1)],
                            preferred_element_type=jnp.float32)
    h3 = jax.nn.relu(acc)
    b_ids = batch_ref[0, 0, :]
    onehot = jnp.where(
        b_ids[:, None] == lax.broadcasted_iota(jnp.int32, (ROW_BLK, N_GRAPHS), 1),
        1.0, 0.0)
    part = lax.dot_general(onehot, h3, (((0,), (0,)), ((), ())),
                           preferred_element_type=jnp.float32)
    pcnt = jnp.sum(onehot, axis=0)[:, None]

    @pl.when(i == 0)
    def _():
        sums_ref[...] = jnp.zeros_like(sums_ref)
        cnt_ref[...] = jnp.zeros_like(cnt_ref)

    sums_ref[...] += part
    cnt_ref[...] += pcnt


def _tc_layer3_pool(ps, hss, dinv, W3, b3, batch3):
    return pl.pallas_call(
        _layer3_body,
        grid=(TC_GRID,),
        in_specs=[
            pl.BlockSpec((2, ROW_BLK, 16), lambda i: (0, i, 0)),
            pl.BlockSpec((2, ROW_BLK, 16), lambda i: (0, i, 0)),
            pl.BlockSpec((2, ROW_BLK, 16), lambda i: (0, i, 0)),
            pl.BlockSpec((2, ROW_BLK, 16), lambda i: (0, i, 0)),
            pl.BlockSpec((ROW_BLK, 16), lambda i: (i, 0)),
            pl.BlockSpec((ROW_BLK, 16), lambda i: (i, 0)),
            pl.BlockSpec((ROW_BLK, 16), lambda i: (i, 0)),
            pl.BlockSpec((ROW_BLK, 16), lambda i: (i, 0)),
            pl.BlockSpec((ROW_BLK, 1), lambda i: (i, 0)),
            pl.BlockSpec((64, 128), lambda i: (0, 0)),
            pl.BlockSpec((1, 128), lambda i: (0, 0)),
            pl.BlockSpec((1, 1, ROW_BLK), lambda i: (i, 0, 0)),
        ],
        out_specs=[
            pl.BlockSpec((N_GRAPHS, 128), lambda i: (0, 0)),
            pl.BlockSpec((N_GRAPHS, 1), lambda i: (0, 0)),
        ],
        out_shape=[
            jax.ShapeDtypeStruct((N_GRAPHS, 128), jnp.float32),
            jax.ShapeDtypeStruct((N_GRAPHS, 1), jnp.float32),
        ],
    )(*ps, *hss, dinv, W3, b3.reshape(1, 128), batch3)


def _final_body(sums_ref, cnt_ref, wl_ref, bl_ref, o_ref):
    pooled = sums_ref[...] / jnp.maximum(cnt_ref[...], 1.0)
    logits = (jnp.dot(pooled, wl_ref[...], preferred_element_type=jnp.float32)
              + bl_ref[...])
    m = jnp.max(logits, axis=1, keepdims=True)
    z = logits - m
    o_ref[...] = z - jnp.log(jnp.sum(jnp.exp(z), axis=1, keepdims=True))


def _tc_final(sums, cnt, Wl, bl):
    nc = Wl.shape[1]
    return pl.pallas_call(
        _final_body,
        grid=(1,),
        in_specs=[
            pl.BlockSpec((N_GRAPHS, 128), lambda i: (0, 0)),
            pl.BlockSpec((N_GRAPHS, 1), lambda i: (0, 0)),
            pl.BlockSpec((128, nc), lambda i: (0, 0)),
            pl.BlockSpec((1, nc), lambda i: (0, 0)),
        ],
        out_specs=pl.BlockSpec((N_GRAPHS, nc), lambda i: (0, 0)),
        out_shape=jax.ShapeDtypeStruct((N_GRAPHS, nc), jnp.float32),
    )(sums, cnt, Wl, bl.reshape(1, nc))


def kernel(x, edge_index, batch, W1, b1, W2, b2, W3, b3, Wl, bl):
    src = edge_index[0]
    dst = edge_index[1]
    src2 = jnp.concatenate(
        [src, jnp.zeros((E_PAD - N_EDGES,), jnp.int32)]).reshape(-1, SUB)
    dst2 = jnp.concatenate(
        [dst, jnp.full((E_PAD - N_EDGES,), N_NODES, jnp.int32)]).reshape(-1, SUB)

    ones = jnp.ones((N_PAD, 16), jnp.float32)
    pdeg = _sc_prop(ones, src2, dst2, fsc=16, do_gather=False)

    xp = jnp.pad(x, ((0, N_PAD - N_NODES), (0, 3)))
    dinv, hs0 = _tc_prep(pdeg, xp)

    p1 = _sc_prop(hs0, src2, dst2, fsc=16)
    W1p = jnp.pad(W1, ((0, 3), (0, 0)))
    (hs1,) = _tc_layer(p1, hs0, dinv, W1p, b1, split=False)

    p2 = _sc_prop(hs1, src2, dst2, fsc=16)
    hs2 = _tc_layer(p2, hs1, dinv, W2, b2, split=True)

    ps = [_sc_prop(h, src2, dst2, fsc=16) for h in hs2]

    batch3 = batch.reshape(TC_GRID, 1, ROW_BLK)
    sums, cnt = _tc_layer3_pool(ps, hs2, dinv, W3, b3, batch3)
    return _tc_final(sums, cnt, Wl, bl)


# speedup vs baseline: 46.5152x; 1.0866x over previous
import functools

import jax
import jax.numpy as jnp
from jax import lax
from jax.experimental import pallas as pl
from jax.experimental.pallas import tpu as pltpu
from jax.experimental.pallas import tpu_sc as plsc

N_NODES = 50000
N_EDGES = 3200000
N_GRAPHS = 512

SUB = 128
NSUB = 8
CHUNK = SUB * NSUB
NW = 32
G_ITERS = 98
E_PAD = NW * CHUNK * G_ITERS
N_PAD = 50048
ROWS_TBL = N_PAD
ROWS_PER_TILE = 3128

ROW_BLK = 2000
TC_GRID = N_NODES // ROW_BLK


def _sc_body(do_gather, hs, src1, dst1, out, idx_s, idx_d, rows, shared,
             sem_g):
    c = lax.axis_index("c")
    s = lax.axis_index("s")
    w = s * 2 + c
    r0 = s * ROWS_PER_TILE
    pltpu.sync_copy(hs.at[pl.ds(r0, ROWS_PER_TILE)],
                    shared.at[pl.ds(r0, ROWS_PER_TILE)])
    if not do_gather:
        pltpu.sync_copy(hs.at[pl.ds(0, CHUNK)], rows)
    plsc.subcore_barrier()

    base = w * G_ITERS * CHUNK

    def body(g, carry):
        e0 = base + g * CHUNK
        pltpu.sync_copy(dst1.at[pl.ds(e0, CHUNK)], idx_d)
        if do_gather:
            pltpu.sync_copy(src1.at[pl.ds(e0, CHUNK)], idx_s)
            pltpu.async_copy(hs.at[idx_s], rows, sem_g).wait()
        pltpu.sync_copy(rows, shared.at[idx_d], add=True)
        return carry

    lax.fori_loop(0, G_ITERS, body, 0)
    plsc.subcore_barrier()
    pltpu.sync_copy(shared.at[pl.ds(r0, ROWS_PER_TILE)],
                    out.at[c].at[pl.ds(r0, ROWS_PER_TILE)])


@functools.partial(jax.jit, static_argnames=("fsc", "do_gather"))
def _sc_prop(hs, src2, dst2, fsc, do_gather=True):
    mesh = plsc.VectorSubcoreMesh(core_axis_name="c", subcore_axis_name="s")
    return pl.kernel(
        functools.partial(_sc_body, do_gather),
        out_type=jax.ShapeDtypeStruct((2, ROWS_TBL, fsc), jnp.float32),
        mesh=mesh,
        scratch_types=[
            pltpu.VMEM((CHUNK,), jnp.int32),
            pltpu.VMEM((CHUNK,), jnp.int32),
            pltpu.VMEM((CHUNK, fsc), jnp.float32),
            pltpu.VMEM_SHARED((ROWS_TBL, fsc), jnp.float32),
            pltpu.SemaphoreType.DMA,
        ],
        compiler_params=pltpu.CompilerParams(use_tc_tiling_on_sc=False),
    )(hs, src2, dst2)


def _prep_body(pdeg_ref, xp_ref, dinv_ref, hs0_ref):
    deg = pdeg_ref[0, :, 0:1] + pdeg_ref[1, :, 0:1] - 1.0
    dinv = lax.rsqrt(deg)
    dinv_ref[...] = dinv
    hs0_ref[...] = xp_ref[...] * dinv


def _tc_prep(pdeg, xp):
    return pl.pallas_call(
        _prep_body,
        grid=(TC_GRID,),
        in_specs=[
            pl.BlockSpec((2, ROW_BLK, 16), lambda i: (0, i, 0)),
            pl.BlockSpec((ROW_BLK, 16), lambda i: (i, 0)),
        ],
        out_specs=[
            pl.BlockSpec((ROW_BLK, 1), lambda i: (i, 0)),
            pl.BlockSpec((ROW_BLK, 16), lambda i: (i, 0)),
        ],
        out_shape=[
            jax.ShapeDtypeStruct((N_NODES, 1), jnp.float32),
            jax.ShapeDtypeStruct((N_PAD, 16), jnp.float32),
        ],
    )(pdeg, xp)


def _layer_body(nout, p_ref, hs_ref, dinv_ref, w_ref, b_ref, *o_refs):
    dinv = dinv_ref[...]
    g = dinv * (p_ref[0] + p_ref[1] - hs_ref[...])
    h = jax.nn.relu(
        jnp.dot(g, w_ref[...], preferred_element_type=jnp.float32)
        + b_ref[...])
    hs_next = h * dinv
    if nout == 1:
        o_refs[0][...] = hs_next
    else:
        for q in range(nout):
            o_refs[q][...] = hs_next[:, 16 * q:16 * (q + 1)]


def _tc_layer(p, hs, dinv, W, b, split):
    fsc = hs.shape[1]
    fout = W.shape[1]
    if split:
        nout = fout // 16
        out_specs = [pl.BlockSpec((ROW_BLK, 16), lambda i: (i, 0))] * nout
        out_shape = [jax.ShapeDtypeStruct((N_PAD, 16), jnp.float32)] * nout
    else:
        nout = 1
        out_specs = [pl.BlockSpec((ROW_BLK, fout), lambda i: (i, 0))]
        out_shape = [jax.ShapeDtypeStruct((N_PAD, fout), jnp.float32)]
    return pl.pallas_call(
        functools.partial(_layer_body, nout),
        grid=(TC_GRID,),
        in_specs=[
            pl.BlockSpec((2, ROW_BLK, fsc), lambda i: (0, i, 0)),
            pl.BlockSpec((ROW_BLK, fsc), lambda i: (i, 0)),
            pl.BlockSpec((ROW_BLK, 1), lambda i: (i, 0)),
            pl.BlockSpec((fsc, fout), lambda i: (0, 0)),
            pl.BlockSpec((1, fout), lambda i: (0, 0)),
        ],
        out_specs=out_specs,
        out_shape=out_shape,
    )(p, hs, dinv, W, b.reshape(1, fout))


def _layer3_body(p0_ref, p1_ref, p2_ref, p3_ref, hs0_ref, hs1_ref, hs2_ref,
                 hs3_ref, dinv_ref, w_ref, b_ref,
                 batch_ref, sums_ref, cnt_ref):
    i = pl.program_id(0)
    dinv = dinv_ref[...]
    w = w_ref[...]
    p_refs = (p0_ref, p1_ref, p2_ref, p3_ref)
    hs_refs = (hs0_ref, hs1_ref, hs2_ref, hs3_ref)
    acc = b_ref[...]
    for q in range(4):
        gq = dinv * (p_refs[q][0] + p_refs[q][1] - hs_refs[q][...])
        acc = acc + jnp.dot(gq, w[16 * q:16 * (q + 1)],
                            preferred_element_type=jnp.float32)
    h3 = jax.nn.relu(acc)
    b_ids = batch_ref[0, 0, :]
    onehot = jnp.where(
        b_ids[:, None] == lax.broadcasted_iota(jnp.int32, (ROW_BLK, N_GRAPHS), 1),
        1.0, 0.0)
    part = lax.dot_general(onehot, h3, (((0,), (0,)), ((), ())),
                           preferred_element_type=jnp.float32)
    pcnt = jnp.sum(onehot, axis=0)[:, None]

    @pl.when(i == 0)
    def _():
        sums_ref[...] = jnp.zeros_like(sums_ref)
        cnt_ref[...] = jnp.zeros_like(cnt_ref)

    sums_ref[...] += part
    cnt_ref[...] += pcnt


def _tc_layer3_pool(ps, hss, dinv, W3, b3, batch3):
    return pl.pallas_call(
        _layer3_body,
        grid=(TC_GRID,),
        in_specs=[
            pl.BlockSpec((2, ROW_BLK, 16), lambda i: (0, i, 0)),
            pl.BlockSpec((2, ROW_BLK, 16), lambda i: (0, i, 0)),
            pl.BlockSpec((2, ROW_BLK, 16), lambda i: (0, i, 0)),
            pl.BlockSpec((2, ROW_BLK, 16), lambda i: (0, i, 0)),
            pl.BlockSpec((ROW_BLK, 16), lambda i: (i, 0)),
            pl.BlockSpec((ROW_BLK, 16), lambda i: (i, 0)),
            pl.BlockSpec((ROW_BLK, 16), lambda i: (i, 0)),
            pl.BlockSpec((ROW_BLK, 16), lambda i: (i, 0)),
            pl.BlockSpec((ROW_BLK, 1), lambda i: (i, 0)),
            pl.BlockSpec((64, 128), lambda i: (0, 0)),
            pl.BlockSpec((1, 128), lambda i: (0, 0)),
            pl.BlockSpec((1, 1, ROW_BLK), lambda i: (i, 0, 0)),
        ],
        out_specs=[
            pl.BlockSpec((N_GRAPHS, 128), lambda i: (0, 0)),
            pl.BlockSpec((N_GRAPHS, 1), lambda i: (0, 0)),
        ],
        out_shape=[
            jax.ShapeDtypeStruct((N_GRAPHS, 128), jnp.float32),
            jax.ShapeDtypeStruct((N_GRAPHS, 1), jnp.float32),
        ],
    )(*ps, *hss, dinv, W3, b3.reshape(1, 128), batch3)


def _final_body(sums_ref, cnt_ref, wl_ref, bl_ref, o_ref):
    pooled = sums_ref[...] / jnp.maximum(cnt_ref[...], 1.0)
    logits = (jnp.dot(pooled, wl_ref[...], preferred_element_type=jnp.float32)
              + bl_ref[...])
    m = jnp.max(logits, axis=1, keepdims=True)
    z = logits - m
    o_ref[...] = z - jnp.log(jnp.sum(jnp.exp(z), axis=1, keepdims=True))


def _tc_final(sums, cnt, Wl, bl):
    nc = Wl.shape[1]
    return pl.pallas_call(
        _final_body,
        grid=(1,),
        in_specs=[
            pl.BlockSpec((N_GRAPHS, 128), lambda i: (0, 0)),
            pl.BlockSpec((N_GRAPHS, 1), lambda i: (0, 0)),
            pl.BlockSpec((128, nc), lambda i: (0, 0)),
            pl.BlockSpec((1, nc), lambda i: (0, 0)),
        ],
        out_specs=pl.BlockSpec((N_GRAPHS, nc), lambda i: (0, 0)),
        out_shape=jax.ShapeDtypeStruct((N_GRAPHS, nc), jnp.float32),
    )(sums, cnt, Wl, bl.reshape(1, nc))


def kernel(x, edge_index, batch, W1, b1, W2, b2, W3, b3, Wl, bl):
    src = edge_index[0]
    dst = edge_index[1]
    src2 = jnp.concatenate(
        [src, jnp.zeros((E_PAD - N_EDGES,), jnp.int32)])
    dst2 = jnp.concatenate(
        [dst, jnp.full((E_PAD - N_EDGES,), N_NODES, jnp.int32)])

    ones = jnp.ones((N_PAD, 16), jnp.float32)
    pdeg = _sc_prop(ones, src2, dst2, fsc=16, do_gather=False)

    xp = jnp.pad(x, ((0, N_PAD - N_NODES), (0, 3)))
    dinv, hs0 = _tc_prep(pdeg, xp)

    p1 = _sc_prop(hs0, src2, dst2, fsc=16)
    W1p = jnp.pad(W1, ((0, 3), (0, 0)))
    (hs1,) = _tc_layer(p1, hs0, dinv, W1p, b1, split=False)

    p2 = _sc_prop(hs1, src2, dst2, fsc=16)
    hs2 = _tc_layer(p2, hs1, dinv, W2, b2, split=True)

    ps = [_sc_prop(h, src2, dst2, fsc=16) for h in hs2]

    batch3 = batch.reshape(TC_GRID, 1, ROW_BLK)
    sums, cnt = _tc_layer3_pool(ps, hs2, dinv, W3, b3, batch3)
    return _tc_final(sums, cnt, Wl, bl)


# speedup vs baseline: 78.5862x; 1.6895x over previous
import functools

import jax
import jax.numpy as jnp
from jax import lax
from jax.experimental import pallas as pl
from jax.experimental.pallas import tpu as pltpu
from jax.experimental.pallas import tpu_sc as plsc

N_NODES = 50000
N_EDGES = 3200000
N_GRAPHS = 512

SUB = 128
NSUB = 8
CHUNK = SUB * NSUB
NW = 32
NBUF = 4
G_ITERS = 98
E_PAD = NW * CHUNK * G_ITERS
N_PAD = 50048
ROWS_TBL = N_PAD
ROWS_PER_TILE = 3128

ROW_BLK = 2000
TC_GRID = N_NODES // ROW_BLK


def _sc_body(do_gather, hs, src1, dst1, out, idx_s, idx_d, rows, shared,
             sem_i, sem_g, sem_s):
    c = lax.axis_index("c")
    s = lax.axis_index("s")
    w = s * 2 + c
    r0 = s * ROWS_PER_TILE
    pltpu.sync_copy(hs.at[pl.ds(r0, ROWS_PER_TILE)],
                    shared.at[pl.ds(r0, ROWS_PER_TILE)])
    if not do_gather:
        pltpu.sync_copy(hs.at[pl.ds(0, CHUNK)], rows.at[0])
    plsc.subcore_barrier()

    base = w * G_ITERS * CHUNK

    def fire_idx(g):
        b = lax.rem(g, NBUF)
        e0 = base + g * CHUNK
        pltpu.async_copy(dst1.at[pl.ds(e0, CHUNK)], idx_d.at[b], sem_i.at[b])
        if do_gather:
            pltpu.async_copy(src1.at[pl.ds(e0, CHUNK)], idx_s.at[b],
                             sem_i.at[b])

    def wait_idx(g):
        b = lax.rem(g, NBUF)
        pltpu.make_async_copy(dst1.at[pl.ds(0, CHUNK)], idx_d.at[b],
                              sem_i.at[b]).wait()
        if do_gather:
            pltpu.make_async_copy(src1.at[pl.ds(0, CHUNK)], idx_s.at[b],
                                  sem_i.at[b]).wait()

    def fire_gather(g):
        if do_gather:
            b = lax.rem(g, NBUF)
            pltpu.async_copy(hs.at[idx_s.at[b]], rows.at[b], sem_g.at[b])

    def wait_gather(g):
        if do_gather:
            b = lax.rem(g, NBUF)
            pltpu.make_async_copy(hs.at[idx_s.at[b]], rows.at[b],
                                  sem_g.at[b]).wait()

    def fire_scatter(g):
        b = lax.rem(g, NBUF)
        rb = b if do_gather else 0
        pltpu.async_copy(rows.at[rb], shared.at[idx_d.at[b]], sem_s.at[b],
                         add=True)

    def wait_scatter(g):
        b = lax.rem(g, NBUF)
        rb = b if do_gather else 0
        pltpu.make_async_copy(rows.at[rb], shared.at[idx_d.at[b]],
                              sem_s.at[b]).wait()

    fire_idx(0)
    fire_idx(1)

    def body(g, carry):
        wait_idx(g)
        fire_gather(g)

        @pl.when(g >= 1)
        def _():
            wait_gather(g - 1)
            fire_scatter(g - 1)

        @pl.when(g >= 2)
        def _():
            wait_scatter(g - 2)

        @pl.when(g + 2 < G_ITERS)
        def _():
            fire_idx(g + 2)

        return carry

    lax.fori_loop(0, G_ITERS, body, 0)
    wait_gather(G_ITERS - 1)
    fire_scatter(G_ITERS - 1)
    wait_scatter(G_ITERS - 2)
    wait_scatter(G_ITERS - 1)
    plsc.subcore_barrier()
    pltpu.sync_copy(shared.at[pl.ds(r0, ROWS_PER_TILE)],
                    out.at[c].at[pl.ds(r0, ROWS_PER_TILE)])


@functools.partial(jax.jit, static_argnames=("fsc", "do_gather"))
def _sc_prop(hs, src2, dst2, fsc, do_gather=True):
    mesh = plsc.VectorSubcoreMesh(core_axis_name="c", subcore_axis_name="s")
    return pl.kernel(
        functools.partial(_sc_body, do_gather),
        out_type=jax.ShapeDtypeStruct((2, ROWS_TBL, fsc), jnp.float32),
        mesh=mesh,
        scratch_types=[
            pltpu.VMEM((NBUF, CHUNK), jnp.int32),
            pltpu.VMEM((NBUF, CHUNK), jnp.int32),
            pltpu.VMEM((NBUF, CHUNK, fsc), jnp.float32),
            pltpu.VMEM_SHARED((ROWS_TBL, fsc), jnp.float32),
            pltpu.SemaphoreType.DMA((NBUF,)),
            pltpu.SemaphoreType.DMA((NBUF,)),
            pltpu.SemaphoreType.DMA((NBUF,)),
        ],
        compiler_params=pltpu.CompilerParams(use_tc_tiling_on_sc=False),
    )(hs, src2, dst2)


def _prep_body(pdeg_ref, xp_ref, dinv_ref, hs0_ref):
    deg = pdeg_ref[0, :, 0:1] + pdeg_ref[1, :, 0:1] - 1.0
    dinv = lax.rsqrt(deg)
    dinv_ref[...] = dinv
    hs0_ref[...] = xp_ref[...] * dinv


def _tc_prep(pdeg, xp):
    return pl.pallas_call(
        _prep_body,
        grid=(TC_GRID,),
        in_specs=[
            pl.BlockSpec((2, ROW_BLK, 16), lambda i: (0, i, 0)),
            pl.BlockSpec((ROW_BLK, 16), lambda i: (i, 0)),
        ],
        out_specs=[
            pl.BlockSpec((ROW_BLK, 1), lambda i: (i, 0)),
            pl.BlockSpec((ROW_BLK, 16), lambda i: (i, 0)),
        ],
        out_shape=[
            jax.ShapeDtypeStruct((N_NODES, 1), jnp.float32),
            jax.ShapeDtypeStruct((N_PAD, 16), jnp.float32),
        ],
    )(pdeg, xp)


def _layer_body(nout, p_ref, hs_ref, dinv_ref, w_ref, b_ref, *o_refs):
    dinv = dinv_ref[...]
    g = dinv * (p_ref[0] + p_ref[1] - hs_ref[...])
    h = jax.nn.relu(
        jnp.dot(g, w_ref[...], preferred_element_type=jnp.float32)
        + b_ref[...])
    hs_next = h * dinv
    if nout == 1:
        o_refs[0][...] = hs_next
    else:
        for q in range(nout):
            o_refs[q][...] = hs_next[:, 16 * q:16 * (q + 1)]


def _tc_layer(p, hs, dinv, W, b, split):
    fsc = hs.shape[1]
    fout = W.shape[1]
    if split:
        nout = fout // 16
        out_specs = [pl.BlockSpec((ROW_BLK, 16), lambda i: (i, 0))] * nout
        out_shape = [jax.ShapeDtypeStruct((N_PAD, 16), jnp.float32)] * nout
    else:
        nout = 1
        out_specs = [pl.BlockSpec((ROW_BLK, fout), lambda i: (i, 0))]
        out_shape = [jax.ShapeDtypeStruct((N_PAD, fout), jnp.float32)]
    return pl.pallas_call(
        functools.partial(_layer_body, nout),
        grid=(TC_GRID,),
        in_specs=[
            pl.BlockSpec((2, ROW_BLK, fsc), lambda i: (0, i, 0)),
            pl.BlockSpec((ROW_BLK, fsc), lambda i: (i, 0)),
            pl.BlockSpec((ROW_BLK, 1), lambda i: (i, 0)),
            pl.BlockSpec((fsc, fout), lambda i: (0, 0)),
            pl.BlockSpec((1, fout), lambda i: (0, 0)),
        ],
        out_specs=out_specs,
        out_shape=out_shape,
    )(p, hs, dinv, W, b.reshape(1, fout))


def _layer3_body(p0_ref, p1_ref, p2_ref, p3_ref, hs0_ref, hs1_ref, hs2_ref,
                 hs3_ref, dinv_ref, w_ref, b_ref,
                 batch_ref, sums_ref, cnt_ref):
    i = pl.program_id(0)
    dinv = dinv_ref[...]
    w = w_ref[...]
    p_refs = (p0_ref, p1_ref, p2_ref, p3_ref)
    hs_refs = (hs0_ref, hs1_ref, hs2_ref, hs3_ref)
    acc = b_ref[...]
    for q in range(4):
        gq = dinv * (p_refs[q][0] + p_refs[q][1] - hs_refs[q][...])
        acc = acc + jnp.dot(gq, w[16 * q:16 * (q + 1)],
                            preferred_element_type=jnp.float32)
    h3 = jax.nn.relu(acc)
    b_ids = batch_ref[0, 0, :]
    onehot = jnp.where(
        b_ids[:, None] == lax.broadcasted_iota(jnp.int32, (ROW_BLK, N_GRAPHS), 1),
        1.0, 0.0)
    part = lax.dot_general(onehot, h3, (((0,), (0,)), ((), ())),
                           preferred_element_type=jnp.float32)
    pcnt = jnp.sum(onehot, axis=0)[:, None]

    @pl.when(i == 0)
    def _():
        sums_ref[...] = jnp.zeros_like(sums_ref)
        cnt_ref[...] = jnp.zeros_like(cnt_ref)

    sums_ref[...] += part
    cnt_ref[...] += pcnt


def _tc_layer3_pool(ps, hss, dinv, W3, b3, batch3):
    return pl.pallas_call(
        _layer3_body,
        grid=(TC_GRID,),
        in_specs=[
            pl.BlockSpec((2, ROW_BLK, 16), lambda i: (0, i, 0)),
            pl.BlockSpec((2, ROW_BLK, 16), lambda i: (0, i, 0)),
            pl.BlockSpec((2, ROW_BLK, 16), lambda i: (0, i, 0)),
            pl.BlockSpec((2, ROW_BLK, 16), lambda i: (0, i, 0)),
            pl.BlockSpec((ROW_BLK, 16), lambda i: (i, 0)),
            pl.BlockSpec((ROW_BLK, 16), lambda i: (i, 0)),
            pl.BlockSpec((ROW_BLK, 16), lambda i: (i, 0)),
            pl.BlockSpec((ROW_BLK, 16), lambda i: (i, 0)),
            pl.BlockSpec((ROW_BLK, 1), lambda i: (i, 0)),
            pl.BlockSpec((64, 128), lambda i: (0, 0)),
            pl.BlockSpec((1, 128), lambda i: (0, 0)),
            pl.BlockSpec((1, 1, ROW_BLK), lambda i: (i, 0, 0)),
        ],
        out_specs=[
            pl.BlockSpec((N_GRAPHS, 128), lambda i: (0, 0)),
            pl.BlockSpec((N_GRAPHS, 1), lambda i: (0, 0)),
        ],
        out_shape=[
            jax.ShapeDtypeStruct((N_GRAPHS, 128), jnp.float32),
            jax.ShapeDtypeStruct((N_GRAPHS, 1), jnp.float32),
        ],
    )(*ps, *hss, dinv, W3, b3.reshape(1, 128), batch3)


def _final_body(sums_ref, cnt_ref, wl_ref, bl_ref, o_ref):
    pooled = sums_ref[...] / jnp.maximum(cnt_ref[...], 1.0)
    logits = (jnp.dot(pooled, wl_ref[...], preferred_element_type=jnp.float32)
              + bl_ref[...])
    m = jnp.max(logits, axis=1, keepdims=True)
    z = logits - m
    o_ref[...] = z - jnp.log(jnp.sum(jnp.exp(z), axis=1, keepdims=True))


def _tc_final(sums, cnt, Wl, bl):
    nc = Wl.shape[1]
    return pl.pallas_call(
        _final_body,
        grid=(1,),
        in_specs=[
            pl.BlockSpec((N_GRAPHS, 128), lambda i: (0, 0)),
            pl.BlockSpec((N_GRAPHS, 1), lambda i: (0, 0)),
            pl.BlockSpec((128, nc), lambda i: (0, 0)),
            pl.BlockSpec((1, nc), lambda i: (0, 0)),
        ],
        out_specs=pl.BlockSpec((N_GRAPHS, nc), lambda i: (0, 0)),
        out_shape=jax.ShapeDtypeStruct((N_GRAPHS, nc), jnp.float32),
    )(sums, cnt, Wl, bl.reshape(1, nc))


def kernel(x, edge_index, batch, W1, b1, W2, b2, W3, b3, Wl, bl):
    src = edge_index[0]
    dst = edge_index[1]
    src2 = jnp.concatenate(
        [src, jnp.zeros((E_PAD - N_EDGES,), jnp.int32)])
    dst2 = jnp.concatenate(
        [dst, jnp.full((E_PAD - N_EDGES,), N_NODES, jnp.int32)])

    ones = jnp.ones((N_PAD, 16), jnp.float32)
    pdeg = _sc_prop(ones, src2, dst2, fsc=16, do_gather=False)

    xp = jnp.pad(x, ((0, N_PAD - N_NODES), (0, 3)))
    dinv, hs0 = _tc_prep(pdeg, xp)

    p1 = _sc_prop(hs0, src2, dst2, fsc=16)
    W1p = jnp.pad(W1, ((0, 3), (0, 0)))
    (hs1,) = _tc_layer(p1, hs0, dinv, W1p, b1, split=False)

    p2 = _sc_prop(hs1, src2, dst2, fsc=16)
    hs2 = _tc_layer(p2, hs1, dinv, W2, b2, split=True)

    ps = [_sc_prop(h, src2, dst2, fsc=16) for h in hs2]

    batch3 = batch.reshape(TC_GRID, 1, ROW_BLK)
    sums, cnt = _tc_layer3_pool(ps, hs2, dinv, W3, b3, batch3)
    return _tc_final(sums, cnt, Wl, bl)
